# Initial kernel scaffold; baseline (speedup 1.0000x reference)
import jax
import jax.numpy as jnp
from jax.experimental import pallas as pl


def kernel(x, edge_index, y, W1_rel, b1_rel, W1_root, W2_rel, b2_rel, W2_root):
    raise NotImplementedError("write your pallas kernel here")


# speedup vs baseline: 11.3368x; 11.3368x over previous
import functools

import jax
import jax.numpy as jnp
from jax import lax
from jax.experimental import pallas as pl
from jax.experimental.pallas import tpu as pltpu
from jax.experimental.pallas import tpu_sc as plsc

N = 10000
D = 128
H = 32
C = 10
E = 320000

NC = 2
NS = 16
NW = NC * NS

CHUNK = 128
NCHUNK = 80
E_PAD = NW * NCHUNK * CHUNK
N_PAD = 10112
RPT = N_PAD // NS
ZERO_ROW = N
TRASH_ROW = N + 8
CW = 16


def _dotT(a, w):
    return lax.dot_general(a, w, (((1,), (1,)), ((), ())),
                           preferred_element_type=jnp.float32)


def _lin1_body(x_ref, wrel_ref, wroot_ref, p_ref, r_ref):
    x = x_ref[...]
    p_ref[...] = _dotT(x, wrel_ref[...])
    r_ref[...] = _dotT(x, wroot_ref[...])


def _mid_body(parts_ref, r_ref, b1_ref, w2rel_ref, w2root_ref, q_ref, s_ref):
    agg = parts_ref[0] + parts_ref[1]
    h = jnp.maximum(agg + b1_ref[...] + r_ref[...], 0.0)
    row = lax.broadcasted_iota(jnp.int32, h.shape, 0)
    h = jnp.where(row < N, h, 0.0)
    q_ref[...] = _dotT(h, w2rel_ref[...])
    s_ref[...] = _dotT(h, w2root_ref[...])


def _loss_body(parts_ref, s_ref, b2_ref, y_ref, out_ref):
    logits = parts_ref[0] + parts_ref[1] + s_ref[...] + b2_ref[...]
    col = lax.broadcasted_iota(jnp.int32, logits.shape, 1)
    lg = jnp.where(col < C, logits, -1e30)
    m = jnp.max(lg, axis=1, keepdims=True)
    ex = jnp.exp(lg - m)
    lse = m + jnp.log(jnp.sum(ex, axis=1, keepdims=True))
    picked = jnp.sum(jnp.where(col == y_ref[...], lg, 0.0), axis=1, keepdims=True)
    nll = lse - picked
    rowi = lax.broadcasted_iota(jnp.int32, nll.shape, 0)
    nll = jnp.where(rowi < N, nll, 0.0)
    out_ref[...] = (jnp.sum(nll) / jnp.float32(N)).reshape(1, 1)


def _make_seg_sum(width):
    mesh = plsc.VectorSubcoreMesh(core_axis_name="c", subcore_axis_name="s",
                                  num_cores=NC, num_subcores=NS)

    @functools.partial(
        pl.kernel,
        out_type=jax.ShapeDtypeStruct((NC, N_PAD, width), jnp.float32),
        mesh=mesh,
        scratch_types=[
            pltpu.VMEM((NCHUNK, CHUNK), jnp.int32),
            pltpu.VMEM((NCHUNK, CHUNK), jnp.int32),
            pltpu.VMEM((CHUNK, width), jnp.float32),
            pltpu.VMEM((CHUNK, width), jnp.float32),
            pltpu.VMEM((RPT, width), jnp.float32),
            pltpu.VMEM_SHARED((N_PAD, width), jnp.float32),
            pltpu.SemaphoreType.DMA,
            pltpu.SemaphoreType.DMA,
        ],
        compiler_params=pltpu.CompilerParams(use_tc_tiling_on_sc=False),
    )
    def seg(src_hbm, dst_hbm, table_hbm, zeros_hbm, out_hbm,
            src_v, dst_v, rows_a, rows_b, zv, acc, sem_a, sem_b):
        c = lax.axis_index("c")
        s = lax.axis_index("s")
        w = c * NS + s

        pltpu.sync_copy(src_hbm.at[w], src_v)
        pltpu.sync_copy(dst_hbm.at[w], dst_v)

        pltpu.sync_copy(zeros_hbm.at[pl.ds(s * RPT, RPT)], zv)
        pltpu.sync_copy(zv, acc.at[pl.ds(s * RPT, RPT)])
        plsc.subcore_barrier()

        pltpu.async_copy(table_hbm.at[src_v.at[0]], rows_a, sem_a)

        def body(jj, carry):
            j0 = 2 * jj
            pltpu.async_copy(table_hbm.at[src_v.at[j0 + 1]], rows_b, sem_b)
            pltpu.make_async_copy(table_hbm.at[src_v.at[j0]], rows_a, sem_a).wait()
            pltpu.sync_copy(rows_a, acc.at[dst_v.at[j0]], add=True)

            @pl.when(jj < NCHUNK // 2 - 1)
            def _():
                pltpu.async_copy(table_hbm.at[src_v.at[j0 + 2]], rows_a, sem_a)

            pltpu.make_async_copy(table_hbm.at[src_v.at[j0 + 1]], rows_b, sem_b).wait()
            pltpu.sync_copy(rows_b, acc.at[dst_v.at[j0 + 1]], add=True)
            return carry

        lax.fori_loop(0, NCHUNK // 2, body, 0)
        plsc.subcore_barrier()

        pltpu.sync_copy(acc.at[pl.ds(s * RPT, RPT)],
                        out_hbm.at[c].at[pl.ds(s * RPT, RPT)])

    return seg


_seg_sum_cache = {}


def _seg_sum(width):
    if width not in _seg_sum_cache:
        _seg_sum_cache[width] = _make_seg_sum(width)
    return _seg_sum_cache[width]


def kernel(x, edge_index, y, W1_rel, b1_rel, W1_root, W2_rel, b2_rel, W2_root):
    f32 = jnp.float32

    x_pad = jnp.pad(x, ((0, N_PAD - N), (0, 0)))
    src = jnp.concatenate(
        [edge_index[0], jnp.full((E_PAD - E,), ZERO_ROW, jnp.int32)]
    ).reshape(NW, NCHUNK, CHUNK)
    dst = jnp.concatenate(
        [edge_index[1], jnp.full((E_PAD - E,), TRASH_ROW, jnp.int32)]
    ).reshape(NW, NCHUNK, CHUNK)
    zeros32 = jnp.zeros((N_PAD, H), f32)
    zeros16 = jnp.zeros((N_PAD, CW), f32)
    w2rel_p = jnp.zeros((CW, H), f32).at[:C].set(W2_rel)
    w2root_p = jnp.zeros((CW, H), f32).at[:C].set(W2_root)
    b1_2d = b1_rel.reshape(1, H)
    b2_2d = jnp.zeros((1, CW), f32).at[0, :C].set(b2_rel)
    y_2d = jnp.pad(y.astype(jnp.int32), (0, N_PAD - N)).reshape(N_PAD, 1)

    p, r = pl.pallas_call(
        _lin1_body,
        out_shape=[jax.ShapeDtypeStruct((N_PAD, H), f32),
                   jax.ShapeDtypeStruct((N_PAD, H), f32)],
    )(x_pad, W1_rel, W1_root)

    parts1 = _seg_sum(H)(src, dst, p, zeros32)

    q, s2 = pl.pallas_call(
        _mid_body,
        out_shape=[jax.ShapeDtypeStruct((N_PAD, CW), f32),
                   jax.ShapeDtypeStruct((N_PAD, CW), f32)],
    )(parts1, r, b1_2d, w2rel_p, w2root_p)

    parts2 = _seg_sum(CW)(src, dst, q, zeros16)

    out = pl.pallas_call(
        _loss_body,
        out_shape=jax.ShapeDtypeStruct((1, 1), f32),
    )(parts2, s2, b2_2d, y_2d)

    return (out[0, 0],)
